# baseline (speedup 1.0000x reference)
import functools

import jax
import jax.numpy as jnp
from jax import lax
from jax.experimental import pallas as pl
from jax.experimental.pallas import tpu as pltpu
from jax.experimental.pallas import tpu_sc as plsc

N = 10000
E = 160000
D = 256
H = 256
NH = 4
C = 64

NPAD = 10240
EPAD = 163840
B = 128
EPT = EPAD // 32
CH = EPT // B
ACC1 = 10016

U0 = 8
U1 = 20 - U0
CHMAX = 4 * max(U0, U1)
EPTMAX = CHMAX * B


def _core_split(c, s):
    u = jnp.where(c == 0, U0, U1)
    ch = u * 4
    cbase = jnp.where(c == 0, 0, U0 * 8192)
    base = cbase + s * (ch * B)
    return base, ch


def _mm_body(x_ref, w_ref, b_ref, o_ref):
    o_ref[...] = (
        jnp.dot(x_ref[...], w_ref[...], preferred_element_type=jnp.float32)
        + b_ref[...]
    )


def _mm(x, w, b, bm=512, bn=128):
    m, k = x.shape
    _, n = w.shape
    bn = min(bn, n)
    return pl.pallas_call(
        _mm_body,
        grid=(m // bm, n // bn),
        in_specs=[
            pl.BlockSpec((bm, k), lambda i, j: (i, 0)),
            pl.BlockSpec((k, bn), lambda i, j: (0, j)),
            pl.BlockSpec((1, bn), lambda i, j: (0, j)),
        ],
        out_specs=pl.BlockSpec((bm, bn), lambda i, j: (i, j)),
        out_shape=jax.ShapeDtypeStruct((m, n), jnp.float32),
    )(x, w, b.reshape(1, -1))


def _mm_groups(x, w, b, ng, gd, bm=512):
    m, k = x.shape
    n = ng * gd
    bn = min(128, n)
    gpb = bn // gd

    def body(x_ref, w_ref, b_ref, o_ref):
        y = (
            jnp.dot(x_ref[...], w_ref[...],
                    preferred_element_type=jnp.float32) + b_ref[...]
        )
        for g2 in range(gpb):
            o_ref[g2] = y[:, g2 * gd:(g2 + 1) * gd]

    return pl.pallas_call(
        body,
        grid=(m // bm, ng // gpb),
        in_specs=[
            pl.BlockSpec((bm, k), lambda i, j: (i, 0)),
            pl.BlockSpec((k, bn), lambda i, j: (0, j)),
            pl.BlockSpec((1, bn), lambda i, j: (0, j)),
        ],
        out_specs=pl.BlockSpec((gpb, bm, gd), lambda i, j: (j, i, 0)),
        out_shape=jax.ShapeDtypeStruct((ng, m, gd), jnp.float32),
    )(x, w, b.reshape(1, -1))


def _relu_cat(agg, res):
    ng, _, _, gd = agg.shape
    with_res = res is not None
    bm = 256

    def body(*refs):
        a_ref = refs[0]
        o_ref = refs[-1]
        for g in range(ng):
            v = a_ref[g, 0] + a_ref[g, 1]
            if with_res:
                v = v + refs[1][:, g * gd:(g + 1) * gd]
            o_ref[:, g * gd:(g + 1) * gd] = jnp.maximum(v, 0.0)

    ins = [agg] + ([res] if with_res else [])
    in_specs = [pl.BlockSpec((ng, 2, bm, gd), lambda i: (0, 0, i, 0))]
    if with_res:
        in_specs.append(pl.BlockSpec((bm, ng * gd), lambda i: (i, 0)))
    return pl.pallas_call(
        body,
        grid=(NPAD // bm,),
        in_specs=in_specs,
        out_specs=pl.BlockSpec((bm, ng * gd), lambda i: (i, 0)),
        out_shape=jax.ShapeDtypeStruct((NPAD, ng * gd), jnp.float32),
    )(*ins)


_MESH = plsc.VectorSubcoreMesh(core_axis_name="c", subcore_axis_name="s")
_SC_PARAMS = pltpu.CompilerParams(use_tc_tiling_on_sc=False,
                                  needs_layout_passes=False)


def _make_pass1():
    @functools.partial(
        pl.kernel,
        mesh=_MESH,
        out_type=[
            jax.ShapeDtypeStruct((EPAD, 16), jnp.float32),
            jax.ShapeDtypeStruct((NPAD, 16), jnp.float32),
            jax.ShapeDtypeStruct((NPAD, 16), jnp.float32),
        ],
        scratch_types=[
            pltpu.VMEM((4, B), jnp.int32),
            pltpu.VMEM((4, B), jnp.int32),
            pltpu.VMEM((4, B, 32), jnp.float32),
            pltpu.VMEM((4, B, 32), jnp.float32),
            pltpu.VMEM((4, B, 16), jnp.float32),
            pltpu.VMEM((32, 16), jnp.float32),
            pltpu.VMEM_SHARED((ACC1, 16), jnp.float32),
            pltpu.SemaphoreType.DMA((4,)),
            pltpu.SemaphoreType.DMA((4,)),
            pltpu.SemaphoreType.DMA((4,)),
        ],
        compiler_params=_SC_PARAMS,
    )
    def pass1(tab, srcp, dstp, e_out, p0_out, p1_out,
              sidx, didx, rd, rs, ev, zb, acc, gsem1, gsem2, ssem):
        c = lax.axis_index("c")
        s = lax.axis_index("s")
        base, ch = _core_split(c, s)

        for i in range(32):
            zb[i, :] = jnp.zeros((16,), jnp.float32)

        def zloop(t, carry):
            blk = t * 16 + s

            @pl.when(blk < ACC1 // 32)
            def _z():
                pltpu.sync_copy(zb, acc.at[pl.ds(blk * 32, 32)])

            return carry

        lax.fori_loop(0, (ACC1 // 32 + 15) // 16, zloop, 0)
        plsc.subcore_barrier()

        def fire(ci, p):
            off = base + ci * B
            pltpu.sync_copy(srcp.at[pl.ds(off, B)], sidx.at[p])
            pltpu.sync_copy(dstp.at[pl.ds(off, B)], didx.at[p])
            pltpu.async_copy(tab.at[didx.at[p]], rd.at[p], gsem1.at[p])
            pltpu.async_copy(tab.at[sidx.at[p]], rs.at[p], gsem2.at[p])

        fire(0, 0)

        def quad(ci4, carry):
            for p in range(4):
                ci = ci4 * 4 + p
                q = (p + 1) % 4

                @pl.when(ci >= 3)
                def _ws():
                    pltpu.make_async_copy(
                        ev.at[q], acc.at[sidx.at[q]], ssem.at[q]).wait()

                @pl.when(ci + 1 < ch)
                def _pf():
                    fire(ci + 1, q)

                pltpu.make_async_copy(tab.at[didx.at[p]], rd.at[p],
                                      gsem1.at[p]).wait()
                pltpu.make_async_copy(tab.at[sidx.at[p]], rs.at[p],
                                      gsem2.at[p]).wait()

                @plsc.parallel_loop(0, B, 1, unroll=4)
                def edge(i):
                    x = rd[p, i, 0:16] + rs[p, i, 16:32]
                    x = jnp.maximum(x, 0.01 * x)
                    ev[p, i, :] = jnp.exp(x)
                off = base + ci * B
                pltpu.sync_copy(ev.at[p], e_out.at[pl.ds(off, B)])
                pltpu.async_copy(ev.at[p], acc.at[sidx.at[p]], ssem.at[p],
                                 add=True)
            return carry

        lax.fori_loop(0, ch // 4, quad, 0)
        for p in range(1, 4):
            pltpu.make_async_copy(ev.at[p], acc.at[sidx.at[p]],
                                  ssem.at[p]).wait()
        plsc.subcore_barrier()

        def cpout(t, carry):
            blk = t * 16 + s

            @pl.when(blk < ACC1 // 32)
            def _cp():
                r = blk * 32

                @pl.when(c == 0)
                def _w0():
                    pltpu.sync_copy(acc.at[pl.ds(r, 32)],
                                    p0_out.at[pl.ds(r, 32)])

                @pl.when(c == 1)
                def _w1():
                    pltpu.sync_copy(acc.at[pl.ds(r, 32)],
                                    p1_out.at[pl.ds(r, 32)])

            return carry

        lax.fori_loop(0, (ACC1 // 32 + 15) // 16, cpout, 0)

    return pass1


def _make_pass2(ngroups, gdim, nheads, nranges):
    jc = gdim // 16
    accrows = ACC1 if nranges == 1 else 5024
    real = N // nranges if nranges > 1 else ACC1
    dummy = real + 8 if nranges > 1 else 0
    cpb = 32 if nranges == 1 else 8
    ncpb = real // cpb
    nzb = accrows // 16

    def scratch_types():
        return [
            pltpu.VMEM((CHMAX, B), jnp.int32),
            pltpu.VMEM((CHMAX, B), jnp.int32),
            pltpu.VMEM((4, B), jnp.int32),
            pltpu.VMEM((2, B, 16), jnp.float32),
            pltpu.VMEM((2, B, 16), jnp.float32),
            pltpu.VMEM((2, B, 16), jnp.float32),
            pltpu.VMEM((nheads, EPTMAX + 16), jnp.float32),
            pltpu.VMEM((4, B, gdim), jnp.float32),
            pltpu.VMEM((16, gdim), jnp.float32),
            pltpu.VMEM_SHARED((accrows, gdim), jnp.float32),
            pltpu.SemaphoreType.DMA((4,)),
            pltpu.SemaphoreType.DMA((4,)),
            pltpu.SemaphoreType.DMA((2,)),
            pltpu.SemaphoreType.DMA((2,)),
        ]

    @functools.partial(
        pl.kernel,
        mesh=_MESH,
        out_type=jax.ShapeDtypeStruct((ngroups, 2, NPAD, gdim), jnp.float32),
        scratch_types=scratch_types(),
        compiler_params=_SC_PARAMS,
    )
    def pass2(*refs):
        fts = refs[:ngroups]
        e_in, p0, p1, srcp, dstp = refs[ngroups:ngroups + 5]
        agg = refs[ngroups + 5]
        (sidx, didx, lidx, ev, r0, r1, ah, rows, zb, acc,
         gsem, ssem, asem1, asem2) = refs[ngroups + 6:]

        c = lax.axis_index("c")
        s = lax.axis_index("s")
        base, ch = _core_split(c, s)
        iota16 = lax.iota(jnp.int32, 16)

        for i in range(16):
            for j in range(jc):
                zb[i, pl.ds(j * 16, 16)] = jnp.zeros((16,), jnp.float32)

        def stage(t, carry):
            off = base + t * B
            pltpu.sync_copy(srcp.at[pl.ds(off, B)], sidx.at[t])
            pltpu.sync_copy(dstp.at[pl.ds(off, B)], didx.at[t])
            return carry

        lax.fori_loop(0, ch, stage, 0)

        def afire(ci, p):
            off = base + ci * B
            pltpu.sync_copy(e_in.at[pl.ds(off, B)], ev.at[p])
            pltpu.async_copy(p0.at[didx.at[ci]], r0.at[p], asem1.at[p])
            pltpu.async_copy(p1.at[didx.at[ci]], r1.at[p], asem2.at[p])

        afire(0, 0)

        def apair(ci2, carry):
            for p in range(2):
                ci = ci2 * 2 + p
                q = 1 - p

                @pl.when(ci + 1 < ch)
                def _pf():
                    afire(ci + 1, q)

                pltpu.make_async_copy(p0.at[didx.at[ci]], r0.at[p],
                                      asem1.at[p]).wait()
                pltpu.make_async_copy(p1.at[didx.at[ci]], r1.at[p],
                                      asem2.at[p]).wait()

                @plsc.parallel_loop(0, B, 1, unroll=4)
                def arow(i):
                    ev[p, i, :] = ev[p, i, :] / (r0[p, i, :] + r1[p, i, :])

                for h in range(nheads):
                    hcol = jnp.full((16,), h, jnp.int32)

                    @plsc.parallel_loop(0, B // 16, 1, unroll=2)
                    def ext(i16):
                        rowi = iota16 + i16 * 16
                        vals = plsc.load_gather(ev.at[p], [rowi, hcol])
                        ah[h, pl.ds(ci * B + i16 * 16, 16)] = vals
            return carry

        lax.fori_loop(0, ch // 2, apair, 0)

        for g in range(ngroups):
            h = g * nheads // ngroups
            for r in range(nranges):
                nbase = r * real

                def zloop(t, carry):
                    blk = t * 16 + s

                    @pl.when(blk < nzb)
                    def _z():
                        pltpu.sync_copy(zb, acc.at[pl.ds(blk * 16, 16)])

                    return carry

                lax.fori_loop(0, (nzb + 15) // 16, zloop, 0)
                plsc.subcore_barrier()

                def bfire(ci, p):
                    pltpu.async_copy(fts[g].at[didx.at[ci]], rows.at[p],
                                     gsem.at[p])

                bfire(0, 0)

                def bquad(ci4, carry):
                    for p in range(4):
                        ci = ci4 * 4 + p
                        q = (p + 1) % 4

                        @pl.when(ci >= 3)
                        def _ws():
                            pltpu.make_async_copy(
                                rows.at[q], acc.at[sidx.at[jnp.int32(0)]],
                                ssem.at[q]).wait()

                        @pl.when(ci + 1 < ch)
                        def _pf():
                            bfire(ci + 1, q)

                        pltpu.make_async_copy(fts[g].at[didx.at[ci]],
                                              rows.at[p], gsem.at[p]).wait()

                        @plsc.parallel_loop(0, B, 1, unroll=2)
                        def scale(i):
                            a16 = ah[h, pl.ds(ci * B + i, 16)]
                            bc = jnp.full((16,), a16[0], jnp.float32)
                            for j in range(jc):
                                rows[p, i, pl.ds(j * 16, 16)] = (
                                    rows[p, i, pl.ds(j * 16, 16)] * bc)

                        if nranges > 1:
                            @plsc.parallel_loop(0, B // 16, 1)
                            def locj(j16):
                                li = sidx[ci, pl.ds(j16 * 16, 16)] - nbase
                                ok = (li >= 0) & (li < real)
                                lidx[p, pl.ds(j16 * 16, 16)] = jnp.where(
                                    ok, li, dummy)

                            pltpu.async_copy(rows.at[p], acc.at[lidx.at[p]],
                                             ssem.at[p], add=True)
                        else:
                            pltpu.async_copy(rows.at[p], acc.at[sidx.at[ci]],
                                             ssem.at[p], add=True)
                    return carry

                lax.fori_loop(0, ch // 4, bquad, 0)
                for p in range(1, 4):
                    pltpu.make_async_copy(rows.at[p],
                                          acc.at[sidx.at[jnp.int32(0)]],
                                          ssem.at[p]).wait()
                plsc.subcore_barrier()

                def cpout(t, carry):
                    blk = t * 16 + s

                    @pl.when(blk < ncpb)
                    def _w():
                        pltpu.sync_copy(
                            acc.at[pl.ds(blk * cpb, cpb)],
                            agg.at[g, c, pl.ds(nbase + blk * cpb, cpb)])

                    return carry

                lax.fori_loop(0, (ncpb + 15) // 16, cpout, 0)
                plsc.subcore_barrier()

    return pass2


_PASS1 = _make_pass1()
_PASS2_H = _make_pass2(16, 64, NH, 1)
_PASS2_F = _make_pass2(4, 16, 1, 1)


def _fold_a(W, b, al, alb, ar, arb):
    nh = W.shape[0]
    k = W.shape[1]
    wal = jnp.einsum('hdk,hk->dh', W, al)
    war = jnp.einsum('hdk,hk->dh', W, ar)
    bal = jnp.einsum('hk,hk->h', b, al) + alb
    bar = jnp.einsum('hk,hk->h', b, ar) + arb
    wa = jnp.zeros((k, 128), jnp.float32)
    wa = wa.at[:, :nh].set(wal).at[:, 16:16 + nh].set(war)
    ba = jnp.zeros((128,), jnp.float32)
    ba = ba.at[:nh].set(bal).at[16:16 + nh].set(bar)
    return wa, ba


def kernel(features, params, src, dst):
    x0 = jnp.pad(features, ((0, NPAD - N), (0, 0)))
    srcp = jnp.concatenate([src, jnp.full((EPAD - E,), N, jnp.int32)])
    dstp = jnp.concatenate([dst, jnp.full((EPAD - E,), N, jnp.int32)])

    def h_layer(x, p, with_res):
        wcat = jnp.concatenate([p['W'][h] for h in range(NH)], axis=1)
        bcat = jnp.concatenate([p['b'][h] for h in range(NH)])
        ft = _mm_groups(x, wcat, bcat, 16, 64)
        wa, ba = _fold_a(p['W'], p['b'], p['al'], p['alb'],
                         p['ar'], p['arb'])
        tab32 = _mm(x, wa, ba)[:, :32]
        e_buf, pa0, pa1 = _PASS1(tab32, srcp, dstp)
        agg = _PASS2_H(*[ft[g] for g in range(16)],
                       e_buf, pa0, pa1, srcp, dstp)
        res = None
        if with_res:
            wrcat = jnp.concatenate([p['Wres'][h] for h in range(NH)], axis=1)
            brcat = jnp.concatenate([p['bres'][h] for h in range(NH)])
            res = _mm(x, wrcat, brcat)
        return _relu_cat(agg, res)

    x1 = h_layer(x0, params['l0'], False)
    x2 = h_layer(x1, params['l1'], True)

    p = params['fin']
    ftf = _mm_groups(x2, p['W'], p['b'], 4, 16)
    resf = _mm(x2, p['Wres'], p['bres'], bn=64)
    wa, ba = _fold_a(p['W'][None], p['b'][None],
                     p['al'][None], p['alb'][None],
                     p['ar'][None], p['arb'][None])
    tab32 = _mm(x2, wa, ba)[:, :32]
    e_buf, pa0, pa1 = _PASS1(tab32, srcp, dstp)
    agg = _PASS2_F(*[ftf[g] for g in range(4)], e_buf, pa0, pa1, srcp, dstp)
    out = _relu_cat(agg, resf)
    return out[:N]


# speedup vs baseline: 1.0757x; 1.0757x over previous
import functools

import jax
import jax.numpy as jnp
from jax import lax
from jax.experimental import pallas as pl
from jax.experimental.pallas import tpu as pltpu
from jax.experimental.pallas import tpu_sc as plsc

N = 10000
E = 160000
D = 256
H = 256
NH = 4
C = 64

NPAD = 10240
EPAD = 163840
B = 128
EPT = EPAD // 32
CH = EPT // B
ACC1 = 10016

U0 = 13
U1 = 20 - U0
CHMAX = 4 * max(U0, U1)
EPTMAX = CHMAX * B


def _core_split(c, s):
    u = jnp.where(c == 0, U0, U1)
    ch = u * 4
    cbase = jnp.where(c == 0, 0, U0 * 8192)
    base = cbase + s * (ch * B)
    return base, ch


def _mm_body(x_ref, w_ref, b_ref, o_ref):
    o_ref[...] = (
        jnp.dot(x_ref[...], w_ref[...], preferred_element_type=jnp.float32)
        + b_ref[...]
    )


def _mm(x, w, b, bm=512, bn=128):
    m, k = x.shape
    _, n = w.shape
    bn = min(bn, n)
    return pl.pallas_call(
        _mm_body,
        grid=(m // bm, n // bn),
        in_specs=[
            pl.BlockSpec((bm, k), lambda i, j: (i, 0)),
            pl.BlockSpec((k, bn), lambda i, j: (0, j)),
            pl.BlockSpec((1, bn), lambda i, j: (0, j)),
        ],
        out_specs=pl.BlockSpec((bm, bn), lambda i, j: (i, j)),
        out_shape=jax.ShapeDtypeStruct((m, n), jnp.float32),
    )(x, w, b.reshape(1, -1))


def _mm_groups(x, w, b, ng, gd, bm=512):
    m, k = x.shape
    n = ng * gd
    bn = min(128, n)
    gpb = bn // gd

    def body(x_ref, w_ref, b_ref, o_ref):
        y = (
            jnp.dot(x_ref[...], w_ref[...],
                    preferred_element_type=jnp.float32) + b_ref[...]
        )
        for g2 in range(gpb):
            o_ref[g2] = y[:, g2 * gd:(g2 + 1) * gd]

    return pl.pallas_call(
        body,
        grid=(m // bm, ng // gpb),
        in_specs=[
            pl.BlockSpec((bm, k), lambda i, j: (i, 0)),
            pl.BlockSpec((k, bn), lambda i, j: (0, j)),
            pl.BlockSpec((1, bn), lambda i, j: (0, j)),
        ],
        out_specs=pl.BlockSpec((gpb, bm, gd), lambda i, j: (j, i, 0)),
        out_shape=jax.ShapeDtypeStruct((ng, m, gd), jnp.float32),
    )(x, w, b.reshape(1, -1))


def _relu_cat(agg, res):
    ng, _, _, gd = agg.shape
    with_res = res is not None
    bm = 256

    def body(*refs):
        a_ref = refs[0]
        o_ref = refs[-1]
        for g in range(ng):
            v = a_ref[g, 0] + a_ref[g, 1]
            if with_res:
                v = v + refs[1][:, g * gd:(g + 1) * gd]
            o_ref[:, g * gd:(g + 1) * gd] = jnp.maximum(v, 0.0)

    ins = [agg] + ([res] if with_res else [])
    in_specs = [pl.BlockSpec((ng, 2, bm, gd), lambda i: (0, 0, i, 0))]
    if with_res:
        in_specs.append(pl.BlockSpec((bm, ng * gd), lambda i: (i, 0)))
    return pl.pallas_call(
        body,
        grid=(NPAD // bm,),
        in_specs=in_specs,
        out_specs=pl.BlockSpec((bm, ng * gd), lambda i: (i, 0)),
        out_shape=jax.ShapeDtypeStruct((NPAD, ng * gd), jnp.float32),
    )(*ins)


_MESH = plsc.VectorSubcoreMesh(core_axis_name="c", subcore_axis_name="s")
_SC_PARAMS = pltpu.CompilerParams(use_tc_tiling_on_sc=False,
                                  needs_layout_passes=False)


def _make_pass1():
    @functools.partial(
        pl.kernel,
        mesh=_MESH,
        out_type=[
            jax.ShapeDtypeStruct((EPAD, 16), jnp.float32),
            jax.ShapeDtypeStruct((NPAD, 16), jnp.float32),
            jax.ShapeDtypeStruct((NPAD, 16), jnp.float32),
        ],
        scratch_types=[
            pltpu.VMEM((4, B), jnp.int32),
            pltpu.VMEM((4, B), jnp.int32),
            pltpu.VMEM((4, B, 32), jnp.float32),
            pltpu.VMEM((4, B, 32), jnp.float32),
            pltpu.VMEM((4, B, 16), jnp.float32),
            pltpu.VMEM((32, 16), jnp.float32),
            pltpu.VMEM_SHARED((ACC1, 16), jnp.float32),
            pltpu.SemaphoreType.DMA((4,)),
            pltpu.SemaphoreType.DMA((4,)),
            pltpu.SemaphoreType.DMA((4,)),
        ],
        compiler_params=_SC_PARAMS,
    )
    def pass1(tab, srcp, dstp, e_out, p0_out, p1_out,
              sidx, didx, rd, rs, ev, zb, acc, gsem1, gsem2, ssem):
        c = lax.axis_index("c")
        s = lax.axis_index("s")
        base, ch = _core_split(c, s)

        for i in range(32):
            zb[i, :] = jnp.zeros((16,), jnp.float32)

        def zloop(t, carry):
            blk = t * 16 + s

            @pl.when(blk < ACC1 // 32)
            def _z():
                pltpu.sync_copy(zb, acc.at[pl.ds(blk * 32, 32)])

            return carry

        lax.fori_loop(0, (ACC1 // 32 + 15) // 16, zloop, 0)
        plsc.subcore_barrier()

        def fire(ci, p):
            off = base + ci * B
            pltpu.sync_copy(srcp.at[pl.ds(off, B)], sidx.at[p])
            pltpu.sync_copy(dstp.at[pl.ds(off, B)], didx.at[p])
            pltpu.async_copy(tab.at[didx.at[p]], rd.at[p], gsem1.at[p])
            pltpu.async_copy(tab.at[sidx.at[p]], rs.at[p], gsem2.at[p])

        fire(0, 0)

        def quad(ci4, carry):
            for p in range(4):
                ci = ci4 * 4 + p
                q = (p + 1) % 4

                @pl.when(ci >= 3)
                def _ws():
                    pltpu.make_async_copy(
                        ev.at[q], acc.at[sidx.at[q]], ssem.at[q]).wait()

                @pl.when(ci + 1 < ch)
                def _pf():
                    fire(ci + 1, q)

                pltpu.make_async_copy(tab.at[didx.at[p]], rd.at[p],
                                      gsem1.at[p]).wait()
                pltpu.make_async_copy(tab.at[sidx.at[p]], rs.at[p],
                                      gsem2.at[p]).wait()

                @plsc.parallel_loop(0, B, 1, unroll=4)
                def edge(i):
                    x = rd[p, i, 0:16] + rs[p, i, 16:32]
                    x = jnp.maximum(x, 0.01 * x)
                    ev[p, i, :] = jnp.exp(x)
                off = base + ci * B
                pltpu.sync_copy(ev.at[p], e_out.at[pl.ds(off, B)])
                pltpu.async_copy(ev.at[p], acc.at[sidx.at[p]], ssem.at[p],
                                 add=True)
            return carry

        lax.fori_loop(0, ch // 4, quad, 0)
        for p in range(1, 4):
            pltpu.make_async_copy(ev.at[p], acc.at[sidx.at[p]],
                                  ssem.at[p]).wait()
        plsc.subcore_barrier()

        def cpout(t, carry):
            blk = t * 16 + s

            @pl.when(blk < ACC1 // 32)
            def _cp():
                r = blk * 32

                @pl.when(c == 0)
                def _w0():
                    pltpu.sync_copy(acc.at[pl.ds(r, 32)],
                                    p0_out.at[pl.ds(r, 32)])

                @pl.when(c == 1)
                def _w1():
                    pltpu.sync_copy(acc.at[pl.ds(r, 32)],
                                    p1_out.at[pl.ds(r, 32)])

            return carry

        lax.fori_loop(0, (ACC1 // 32 + 15) // 16, cpout, 0)

    return pass1


def _make_pass2(ngroups, gdim, nheads, nranges):
    jc = gdim // 16
    accrows = ACC1 if nranges == 1 else 5024
    real = N // nranges if nranges > 1 else ACC1
    dummy = real + 8 if nranges > 1 else 0
    cpb = 32 if nranges == 1 else 8
    ncpb = real // cpb
    nzb = accrows // 16

    def scratch_types():
        return [
            pltpu.VMEM((CHMAX, B), jnp.int32),
            pltpu.VMEM((CHMAX, B), jnp.int32),
            pltpu.VMEM((4, B), jnp.int32),
            pltpu.VMEM((2, B, 16), jnp.float32),
            pltpu.VMEM((2, B, 16), jnp.float32),
            pltpu.VMEM((2, B, 16), jnp.float32),
            pltpu.VMEM((nheads, EPTMAX + 16), jnp.float32),
            pltpu.VMEM((4, B, gdim), jnp.float32),
            pltpu.VMEM((16, gdim), jnp.float32),
            pltpu.VMEM_SHARED((accrows, gdim), jnp.float32),
            pltpu.SemaphoreType.DMA((4,)),
            pltpu.SemaphoreType.DMA((4,)),
            pltpu.SemaphoreType.DMA((2,)),
            pltpu.SemaphoreType.DMA((2,)),
        ]

    @functools.partial(
        pl.kernel,
        mesh=_MESH,
        out_type=jax.ShapeDtypeStruct((ngroups, 2, NPAD, gdim), jnp.float32),
        scratch_types=scratch_types(),
        compiler_params=_SC_PARAMS,
    )
    def pass2(*refs):
        fts = refs[:ngroups]
        e_in, p0, p1, srcp, dstp = refs[ngroups:ngroups + 5]
        agg = refs[ngroups + 5]
        (sidx, didx, lidx, ev, r0, r1, ah, rows, zb, acc,
         gsem, ssem, asem1, asem2) = refs[ngroups + 6:]

        c = lax.axis_index("c")
        s = lax.axis_index("s")
        base, ch = _core_split(c, s)
        iota16 = lax.iota(jnp.int32, 16)

        for i in range(16):
            for j in range(jc):
                zb[i, pl.ds(j * 16, 16)] = jnp.zeros((16,), jnp.float32)

        def stage(t, carry):
            off = base + t * B
            pltpu.sync_copy(srcp.at[pl.ds(off, B)], sidx.at[t])
            pltpu.sync_copy(dstp.at[pl.ds(off, B)], didx.at[t])
            return carry

        lax.fori_loop(0, ch, stage, 0)

        def afire(ci, p):
            off = base + ci * B
            pltpu.sync_copy(e_in.at[pl.ds(off, B)], ev.at[p])
            pltpu.async_copy(p0.at[didx.at[ci]], r0.at[p], asem1.at[p])
            pltpu.async_copy(p1.at[didx.at[ci]], r1.at[p], asem2.at[p])

        afire(0, 0)

        def apair(ci2, carry):
            for p in range(2):
                ci = ci2 * 2 + p
                q = 1 - p

                @pl.when(ci + 1 < ch)
                def _pf():
                    afire(ci + 1, q)

                pltpu.make_async_copy(p0.at[didx.at[ci]], r0.at[p],
                                      asem1.at[p]).wait()
                pltpu.make_async_copy(p1.at[didx.at[ci]], r1.at[p],
                                      asem2.at[p]).wait()

                @plsc.parallel_loop(0, B, 1, unroll=4)
                def arow(i):
                    ev[p, i, :] = ev[p, i, :] / (r0[p, i, :] + r1[p, i, :])

                for h in range(nheads):
                    hcol = jnp.full((16,), h, jnp.int32)

                    @plsc.parallel_loop(0, B // 16, 1, unroll=2)
                    def ext(i16):
                        rowi = iota16 + i16 * 16
                        vals = plsc.load_gather(ev.at[p], [rowi, hcol])
                        ah[h, pl.ds(ci * B + i16 * 16, 16)] = vals
            return carry

        lax.fori_loop(0, ch // 2, apair, 0)

        for g in range(ngroups):
            h = g * nheads // ngroups
            for r in range(nranges):
                nbase = r * real

                def zloop(t, carry):
                    blk = t * 16 + s

                    @pl.when(blk < nzb)
                    def _z():
                        pltpu.sync_copy(zb, acc.at[pl.ds(blk * 16, 16)])

                    return carry

                lax.fori_loop(0, (nzb + 15) // 16, zloop, 0)
                plsc.subcore_barrier()

                def bfire(ci, p):
                    pltpu.async_copy(fts[g].at[didx.at[ci]], rows.at[p],
                                     gsem.at[p])

                bfire(0, 0)

                def bquad(ci4, carry):
                    for p in range(4):
                        ci = ci4 * 4 + p
                        q = (p + 1) % 4

                        @pl.when(ci >= 3)
                        def _ws():
                            pltpu.make_async_copy(
                                rows.at[q], acc.at[sidx.at[jnp.int32(0)]],
                                ssem.at[q]).wait()

                        @pl.when(ci + 1 < ch)
                        def _pf():
                            bfire(ci + 1, q)

                        pltpu.make_async_copy(fts[g].at[didx.at[ci]],
                                              rows.at[p], gsem.at[p]).wait()

                        @plsc.parallel_loop(0, B, 1, unroll=2)
                        def scale(i):
                            a16 = ah[h, pl.ds(ci * B + i, 16)]
                            bc = jnp.full((16,), a16[0], jnp.float32)
                            for j in range(jc):
                                rows[p, i, pl.ds(j * 16, 16)] = (
                                    rows[p, i, pl.ds(j * 16, 16)] * bc)

                        if nranges > 1:
                            @plsc.parallel_loop(0, B // 16, 1)
                            def locj(j16):
                                li = sidx[ci, pl.ds(j16 * 16, 16)] - nbase
                                ok = (li >= 0) & (li < real)
                                lidx[p, pl.ds(j16 * 16, 16)] = jnp.where(
                                    ok, li, dummy)

                            pltpu.async_copy(rows.at[p], acc.at[lidx.at[p]],
                                             ssem.at[p], add=True)
                        else:
                            pltpu.async_copy(rows.at[p], acc.at[sidx.at[ci]],
                                             ssem.at[p], add=True)
                    return carry

                lax.fori_loop(0, ch // 4, bquad, 0)
                for p in range(1, 4):
                    pltpu.make_async_copy(rows.at[p],
                                          acc.at[sidx.at[jnp.int32(0)]],
                                          ssem.at[p]).wait()
                plsc.subcore_barrier()

                def cpout(t, carry):
                    blk = t * 16 + s

                    @pl.when(blk < ncpb)
                    def _w():
                        pltpu.sync_copy(
                            acc.at[pl.ds(blk * cpb, cpb)],
                            agg.at[g, c, pl.ds(nbase + blk * cpb, cpb)])

                    return carry

                lax.fori_loop(0, (ncpb + 15) // 16, cpout, 0)
                plsc.subcore_barrier()

    return pass2


_PASS1 = _make_pass1()
_PASS2_H = _make_pass2(16, 64, NH, 1)
_PASS2_F = _make_pass2(4, 16, 1, 1)


def _fold_a(W, b, al, alb, ar, arb):
    nh = W.shape[0]
    k = W.shape[1]
    wal = jnp.einsum('hdk,hk->dh', W, al)
    war = jnp.einsum('hdk,hk->dh', W, ar)
    bal = jnp.einsum('hk,hk->h', b, al) + alb
    bar = jnp.einsum('hk,hk->h', b, ar) + arb
    wa = jnp.zeros((k, 128), jnp.float32)
    wa = wa.at[:, :nh].set(wal).at[:, 16:16 + nh].set(war)
    ba = jnp.zeros((128,), jnp.float32)
    ba = ba.at[:nh].set(bal).at[16:16 + nh].set(bar)
    return wa, ba


def kernel(features, params, src, dst):
    x0 = jnp.pad(features, ((0, NPAD - N), (0, 0)))
    srcp = jnp.concatenate([src, jnp.full((EPAD - E,), N, jnp.int32)])
    dstp = jnp.concatenate([dst, jnp.full((EPAD - E,), N, jnp.int32)])

    def h_layer(x, p, with_res):
        wcat = jnp.concatenate([p['W'][h] for h in range(NH)], axis=1)
        bcat = jnp.concatenate([p['b'][h] for h in range(NH)])
        ft = _mm_groups(x, wcat, bcat, 16, 64)
        wa, ba = _fold_a(p['W'], p['b'], p['al'], p['alb'],
                         p['ar'], p['arb'])
        tab32 = _mm(x, wa, ba)[:, :32]
        e_buf, pa0, pa1 = _PASS1(tab32, srcp, dstp)
        agg = _PASS2_H(*[ft[g] for g in range(16)],
                       e_buf, pa0, pa1, srcp, dstp)
        res = None
        if with_res:
            wrcat = jnp.concatenate([p['Wres'][h] for h in range(NH)], axis=1)
            brcat = jnp.concatenate([p['bres'][h] for h in range(NH)])
            res = _mm(x, wrcat, brcat)
        return _relu_cat(agg, res)

    x1 = h_layer(x0, params['l0'], False)
    x2 = h_layer(x1, params['l1'], True)

    p = params['fin']
    ftf = _mm_groups(x2, p['W'], p['b'], 4, 16)
    resf = _mm(x2, p['Wres'], p['bres'], bn=64)
    wa, ba = _fold_a(p['W'][None], p['b'][None],
                     p['al'][None], p['alb'][None],
                     p['ar'][None], p['arb'][None])
    tab32 = _mm(x2, wa, ba)[:, :32]
    e_buf, pa0, pa1 = _PASS1(tab32, srcp, dstp)
    agg = _PASS2_F(*[ftf[g] for g in range(4)], e_buf, pa0, pa1, srcp, dstp)
    out = _relu_cat(agg, resf)
    return out[:N]


# speedup vs baseline: 1.0794x; 1.0035x over previous
import functools

import jax
import jax.numpy as jnp
from jax import lax
from jax.experimental import pallas as pl
from jax.experimental.pallas import tpu as pltpu
from jax.experimental.pallas import tpu_sc as plsc

N = 10000
E = 160000
D = 256
H = 256
NH = 4
C = 64

NPAD = 10240
EPAD = 163840
B = 128
EPT = EPAD // 32
CH = EPT // B
ACC1 = 10016

U0 = 14
U1 = 20 - U0
CHMAX = 4 * max(U0, U1)
EPTMAX = CHMAX * B


def _core_split(c, s):
    u = jnp.where(c == 0, U0, U1)
    ch = u * 4
    cbase = jnp.where(c == 0, 0, U0 * 8192)
    base = cbase + s * (ch * B)
    return base, ch


def _mm_body(x_ref, w_ref, b_ref, o_ref):
    o_ref[...] = (
        jnp.dot(x_ref[...], w_ref[...], preferred_element_type=jnp.float32)
        + b_ref[...]
    )


def _mm(x, w, b, bm=512, bn=128):
    m, k = x.shape
    _, n = w.shape
    bn = min(bn, n)
    return pl.pallas_call(
        _mm_body,
        grid=(m // bm, n // bn),
        in_specs=[
            pl.BlockSpec((bm, k), lambda i, j: (i, 0)),
            pl.BlockSpec((k, bn), lambda i, j: (0, j)),
            pl.BlockSpec((1, bn), lambda i, j: (0, j)),
        ],
        out_specs=pl.BlockSpec((bm, bn), lambda i, j: (i, j)),
        out_shape=jax.ShapeDtypeStruct((m, n), jnp.float32),
    )(x, w, b.reshape(1, -1))


def _mm_groups(x, w, b, ng, gd, bm=512):
    m, k = x.shape
    n = ng * gd
    bn = min(128, n)
    gpb = bn // gd

    def body(x_ref, w_ref, b_ref, o_ref):
        y = (
            jnp.dot(x_ref[...], w_ref[...],
                    preferred_element_type=jnp.float32) + b_ref[...]
        )
        for g2 in range(gpb):
            o_ref[g2] = y[:, g2 * gd:(g2 + 1) * gd]

    return pl.pallas_call(
        body,
        grid=(m // bm, ng // gpb),
        in_specs=[
            pl.BlockSpec((bm, k), lambda i, j: (i, 0)),
            pl.BlockSpec((k, bn), lambda i, j: (0, j)),
            pl.BlockSpec((1, bn), lambda i, j: (0, j)),
        ],
        out_specs=pl.BlockSpec((gpb, bm, gd), lambda i, j: (j, i, 0)),
        out_shape=jax.ShapeDtypeStruct((ng, m, gd), jnp.float32),
    )(x, w, b.reshape(1, -1))


def _relu_cat(agg, res):
    ng, _, _, gd = agg.shape
    with_res = res is not None
    bm = 256

    def body(*refs):
        a_ref = refs[0]
        o_ref = refs[-1]
        for g in range(ng):
            v = a_ref[g, 0] + a_ref[g, 1]
            if with_res:
                v = v + refs[1][:, g * gd:(g + 1) * gd]
            o_ref[:, g * gd:(g + 1) * gd] = jnp.maximum(v, 0.0)

    ins = [agg] + ([res] if with_res else [])
    in_specs = [pl.BlockSpec((ng, 2, bm, gd), lambda i: (0, 0, i, 0))]
    if with_res:
        in_specs.append(pl.BlockSpec((bm, ng * gd), lambda i: (i, 0)))
    return pl.pallas_call(
        body,
        grid=(NPAD // bm,),
        in_specs=in_specs,
        out_specs=pl.BlockSpec((bm, ng * gd), lambda i: (i, 0)),
        out_shape=jax.ShapeDtypeStruct((NPAD, ng * gd), jnp.float32),
    )(*ins)


_MESH = plsc.VectorSubcoreMesh(core_axis_name="c", subcore_axis_name="s")
_SC_PARAMS = pltpu.CompilerParams(use_tc_tiling_on_sc=False,
                                  needs_layout_passes=False)


def _make_pass1():
    @functools.partial(
        pl.kernel,
        mesh=_MESH,
        out_type=[
            jax.ShapeDtypeStruct((EPAD, 16), jnp.float32),
            jax.ShapeDtypeStruct((NPAD, 16), jnp.float32),
            jax.ShapeDtypeStruct((NPAD, 16), jnp.float32),
        ],
        scratch_types=[
            pltpu.VMEM((4, B), jnp.int32),
            pltpu.VMEM((4, B), jnp.int32),
            pltpu.VMEM((4, B, 32), jnp.float32),
            pltpu.VMEM((4, B, 32), jnp.float32),
            pltpu.VMEM((4, B, 16), jnp.float32),
            pltpu.VMEM((32, 16), jnp.float32),
            pltpu.VMEM_SHARED((ACC1, 16), jnp.float32),
            pltpu.SemaphoreType.DMA((4,)),
            pltpu.SemaphoreType.DMA((4,)),
            pltpu.SemaphoreType.DMA((4,)),
        ],
        compiler_params=_SC_PARAMS,
    )
    def pass1(tab, srcp, dstp, e_out, p0_out, p1_out,
              sidx, didx, rd, rs, ev, zb, acc, gsem1, gsem2, ssem):
        c = lax.axis_index("c")
        s = lax.axis_index("s")
        base, ch = _core_split(c, s)

        for i in range(32):
            zb[i, :] = jnp.zeros((16,), jnp.float32)

        def zloop(t, carry):
            blk = t * 16 + s

            @pl.when(blk < ACC1 // 32)
            def _z():
                pltpu.sync_copy(zb, acc.at[pl.ds(blk * 32, 32)])

            return carry

        lax.fori_loop(0, (ACC1 // 32 + 15) // 16, zloop, 0)
        plsc.subcore_barrier()

        def fire(ci, p):
            off = base + ci * B
            pltpu.sync_copy(srcp.at[pl.ds(off, B)], sidx.at[p])
            pltpu.sync_copy(dstp.at[pl.ds(off, B)], didx.at[p])
            pltpu.async_copy(tab.at[didx.at[p]], rd.at[p], gsem1.at[p])
            pltpu.async_copy(tab.at[sidx.at[p]], rs.at[p], gsem2.at[p])

        fire(0, 0)

        def quad(ci4, carry):
            for p in range(4):
                ci = ci4 * 4 + p
                q = (p + 1) % 4

                @pl.when(ci >= 3)
                def _ws():
                    pltpu.make_async_copy(
                        ev.at[q], acc.at[sidx.at[q]], ssem.at[q]).wait()

                @pl.when(ci + 1 < ch)
                def _pf():
                    fire(ci + 1, q)

                pltpu.make_async_copy(tab.at[didx.at[p]], rd.at[p],
                                      gsem1.at[p]).wait()
                pltpu.make_async_copy(tab.at[sidx.at[p]], rs.at[p],
                                      gsem2.at[p]).wait()

                @plsc.parallel_loop(0, B, 1, unroll=4)
                def edge(i):
                    x = rd[p, i, 0:16] + rs[p, i, 16:32]
                    x = jnp.maximum(x, 0.01 * x)
                    ev[p, i, :] = jnp.exp(x)
                off = base + ci * B
                pltpu.sync_copy(ev.at[p], e_out.at[pl.ds(off, B)])
                pltpu.async_copy(ev.at[p], acc.at[sidx.at[p]], ssem.at[p],
                                 add=True)
            return carry

        lax.fori_loop(0, ch // 4, quad, 0)
        for p in range(1, 4):
            pltpu.make_async_copy(ev.at[p], acc.at[sidx.at[p]],
                                  ssem.at[p]).wait()
        plsc.subcore_barrier()

        def cpout(t, carry):
            blk = t * 16 + s

            @pl.when(blk < ACC1 // 32)
            def _cp():
                r = blk * 32

                @pl.when(c == 0)
                def _w0():
                    pltpu.sync_copy(acc.at[pl.ds(r, 32)],
                                    p0_out.at[pl.ds(r, 32)])

                @pl.when(c == 1)
                def _w1():
                    pltpu.sync_copy(acc.at[pl.ds(r, 32)],
                                    p1_out.at[pl.ds(r, 32)])

            return carry

        lax.fori_loop(0, (ACC1 // 32 + 15) // 16, cpout, 0)

    return pass1


def _make_pass2(ngroups, gdim, nheads, nranges):
    jc = gdim // 16
    accrows = ACC1 if nranges == 1 else 5024
    real = N // nranges if nranges > 1 else ACC1
    dummy = real + 8 if nranges > 1 else 0
    cpb = 32 if nranges == 1 else 8
    ncpb = real // cpb
    nzb = accrows // 16

    def scratch_types():
        return [
            pltpu.VMEM((CHMAX, B), jnp.int32),
            pltpu.VMEM((CHMAX, B), jnp.int32),
            pltpu.VMEM((4, B), jnp.int32),
            pltpu.VMEM((2, B, 16), jnp.float32),
            pltpu.VMEM((2, B, 16), jnp.float32),
            pltpu.VMEM((2, B, 16), jnp.float32),
            pltpu.VMEM((nheads, EPTMAX + 16), jnp.float32),
            pltpu.VMEM((4, B, gdim), jnp.float32),
            pltpu.VMEM((16, gdim), jnp.float32),
            pltpu.VMEM_SHARED((accrows, gdim), jnp.float32),
            pltpu.SemaphoreType.DMA((4,)),
            pltpu.SemaphoreType.DMA((4,)),
            pltpu.SemaphoreType.DMA((2,)),
            pltpu.SemaphoreType.DMA((2,)),
        ]

    @functools.partial(
        pl.kernel,
        mesh=_MESH,
        out_type=jax.ShapeDtypeStruct((ngroups, 2, NPAD, gdim), jnp.float32),
        scratch_types=scratch_types(),
        compiler_params=_SC_PARAMS,
    )
    def pass2(*refs):
        fts = refs[:ngroups]
        e_in, p0, p1, srcp, dstp = refs[ngroups:ngroups + 5]
        agg = refs[ngroups + 5]
        (sidx, didx, lidx, ev, r0, r1, ah, rows, zb, acc,
         gsem, ssem, asem1, asem2) = refs[ngroups + 6:]

        c = lax.axis_index("c")
        s = lax.axis_index("s")
        base, ch = _core_split(c, s)
        iota16 = lax.iota(jnp.int32, 16)

        for i in range(16):
            for j in range(jc):
                zb[i, pl.ds(j * 16, 16)] = jnp.zeros((16,), jnp.float32)

        def stage(t, carry):
            off = base + t * B
            pltpu.sync_copy(srcp.at[pl.ds(off, B)], sidx.at[t])
            pltpu.sync_copy(dstp.at[pl.ds(off, B)], didx.at[t])
            return carry

        lax.fori_loop(0, ch, stage, 0)

        def afire(ci, p):
            off = base + ci * B
            pltpu.sync_copy(e_in.at[pl.ds(off, B)], ev.at[p])
            pltpu.async_copy(p0.at[didx.at[ci]], r0.at[p], asem1.at[p])
            pltpu.async_copy(p1.at[didx.at[ci]], r1.at[p], asem2.at[p])

        afire(0, 0)

        def apair(ci2, carry):
            for p in range(2):
                ci = ci2 * 2 + p
                q = 1 - p

                @pl.when(ci + 1 < ch)
                def _pf():
                    afire(ci + 1, q)

                pltpu.make_async_copy(p0.at[didx.at[ci]], r0.at[p],
                                      asem1.at[p]).wait()
                pltpu.make_async_copy(p1.at[didx.at[ci]], r1.at[p],
                                      asem2.at[p]).wait()

                @plsc.parallel_loop(0, B, 1, unroll=4)
                def arow(i):
                    ev[p, i, :] = ev[p, i, :] / (r0[p, i, :] + r1[p, i, :])

                for h in range(nheads):
                    hcol = jnp.full((16,), h, jnp.int32)

                    @plsc.parallel_loop(0, B // 16, 1, unroll=2)
                    def ext(i16):
                        rowi = iota16 + i16 * 16
                        vals = plsc.load_gather(ev.at[p], [rowi, hcol])
                        ah[h, pl.ds(ci * B + i16 * 16, 16)] = vals
            return carry

        lax.fori_loop(0, ch // 2, apair, 0)

        for g in range(ngroups):
            h = g * nheads // ngroups
            for r in range(nranges):
                nbase = r * real

                def zloop(t, carry):
                    blk = t * 16 + s

                    @pl.when(blk < nzb)
                    def _z():
                        pltpu.sync_copy(zb, acc.at[pl.ds(blk * 16, 16)])

                    return carry

                lax.fori_loop(0, (nzb + 15) // 16, zloop, 0)
                plsc.subcore_barrier()

                def bfire(ci, p):
                    pltpu.async_copy(fts[g].at[didx.at[ci]], rows.at[p],
                                     gsem.at[p])

                bfire(0, 0)

                def bquad(ci4, carry):
                    for p in range(4):
                        ci = ci4 * 4 + p
                        q = (p + 1) % 4

                        @pl.when(ci >= 3)
                        def _ws():
                            pltpu.make_async_copy(
                                rows.at[q], acc.at[sidx.at[jnp.int32(0)]],
                                ssem.at[q]).wait()

                        @pl.when(ci + 1 < ch)
                        def _pf():
                            bfire(ci + 1, q)

                        pltpu.make_async_copy(fts[g].at[didx.at[ci]],
                                              rows.at[p], gsem.at[p]).wait()

                        @plsc.parallel_loop(0, B, 1, unroll=2)
                        def scale(i):
                            a16 = ah[h, pl.ds(ci * B + i, 16)]
                            bc = jnp.full((16,), a16[0], jnp.float32)
                            for j in range(jc):
                                rows[p, i, pl.ds(j * 16, 16)] = (
                                    rows[p, i, pl.ds(j * 16, 16)] * bc)

                        if nranges > 1:
                            @plsc.parallel_loop(0, B // 16, 1)
                            def locj(j16):
                                li = sidx[ci, pl.ds(j16 * 16, 16)] - nbase
                                ok = (li >= 0) & (li < real)
                                lidx[p, pl.ds(j16 * 16, 16)] = jnp.where(
                                    ok, li, dummy)

                            pltpu.async_copy(rows.at[p], acc.at[lidx.at[p]],
                                             ssem.at[p], add=True)
                        else:
                            pltpu.async_copy(rows.at[p], acc.at[sidx.at[ci]],
                                             ssem.at[p], add=True)
                    return carry

                lax.fori_loop(0, ch // 4, bquad, 0)
                for p in range(1, 4):
                    pltpu.make_async_copy(rows.at[p],
                                          acc.at[sidx.at[jnp.int32(0)]],
                                          ssem.at[p]).wait()
                plsc.subcore_barrier()

                def cpout(t, carry):
                    blk = t * 16 + s

                    @pl.when(blk < ncpb)
                    def _w():
                        pltpu.sync_copy(
                            acc.at[pl.ds(blk * cpb, cpb)],
                            agg.at[g, c, pl.ds(nbase + blk * cpb, cpb)])

                    return carry

                lax.fori_loop(0, (ncpb + 15) // 16, cpout, 0)
                plsc.subcore_barrier()

    return pass2


_PASS1 = _make_pass1()
_PASS2_H = _make_pass2(16, 64, NH, 1)
_PASS2_F = _make_pass2(4, 16, 1, 1)


def _fold_a(W, b, al, alb, ar, arb):
    nh = W.shape[0]
    k = W.shape[1]
    wal = jnp.einsum('hdk,hk->dh', W, al)
    war = jnp.einsum('hdk,hk->dh', W, ar)
    bal = jnp.einsum('hk,hk->h', b, al) + alb
    bar = jnp.einsum('hk,hk->h', b, ar) + arb
    wa = jnp.zeros((k, 128), jnp.float32)
    wa = wa.at[:, :nh].set(wal).at[:, 16:16 + nh].set(war)
    ba = jnp.zeros((128,), jnp.float32)
    ba = ba.at[:nh].set(bal).at[16:16 + nh].set(bar)
    return wa, ba


def kernel(features, params, src, dst):
    x0 = jnp.pad(features, ((0, NPAD - N), (0, 0)))
    srcp = jnp.concatenate([src, jnp.full((EPAD - E,), N, jnp.int32)])
    dstp = jnp.concatenate([dst, jnp.full((EPAD - E,), N, jnp.int32)])

    def h_layer(x, p, with_res):
        wcat = jnp.concatenate([p['W'][h] for h in range(NH)], axis=1)
        bcat = jnp.concatenate([p['b'][h] for h in range(NH)])
        ft = _mm_groups(x, wcat, bcat, 16, 64)
        wa, ba = _fold_a(p['W'], p['b'], p['al'], p['alb'],
                         p['ar'], p['arb'])
        tab32 = _mm(x, wa, ba)[:, :32]
        e_buf, pa0, pa1 = _PASS1(tab32, srcp, dstp)
        agg = _PASS2_H(*[ft[g] for g in range(16)],
                       e_buf, pa0, pa1, srcp, dstp)
        res = None
        if with_res:
            wrcat = jnp.concatenate([p['Wres'][h] for h in range(NH)], axis=1)
            brcat = jnp.concatenate([p['bres'][h] for h in range(NH)])
            res = _mm(x, wrcat, brcat)
        return _relu_cat(agg, res)

    x1 = h_layer(x0, params['l0'], False)
    x2 = h_layer(x1, params['l1'], True)

    p = params['fin']
    ftf = _mm_groups(x2, p['W'], p['b'], 4, 16)
    resf = _mm(x2, p['Wres'], p['bres'], bn=64)
    wa, ba = _fold_a(p['W'][None], p['b'][None],
                     p['al'][None], p['alb'][None],
                     p['ar'][None], p['arb'][None])
    tab32 = _mm(x2, wa, ba)[:, :32]
    e_buf, pa0, pa1 = _PASS1(tab32, srcp, dstp)
    agg = _PASS2_F(*[ftf[g] for g in range(4)], e_buf, pa0, pa1, srcp, dstp)
    out = _relu_cat(agg, resf)
    return out[:N]
